# baseline (speedup 1.0000x reference)
import jax
import jax.numpy as jnp
import numpy as np
from jax.experimental import pallas as pl
from jax.experimental.pallas import tpu as pltpu

_M = 1024
_K = 32768
_R = 16
_C = 512
_NC = _K // _C
_TINY = np.float32(np.finfo(np.float32).tiny)


def _rotl(x, r):
    return (x << jnp.uint32(r)) | (x >> jnp.uint32(32 - r))


def _threefry_bits(cnt):
    k0 = jnp.uint32(0)
    k1 = jnp.uint32(42)
    k2 = k0 ^ k1 ^ jnp.uint32(0x1BD11BDA)
    ks = (k0, k1, k2)
    rot = ((13, 15, 26, 6), (17, 29, 16, 24))
    x1 = cnt + ks[1]
    x0 = x1
    x1 = _rotl(x1, 13)
    x1 = x1 ^ x0
    for r in (15, 26, 6):
        x0 = x0 + x1
        x1 = _rotl(x1, r)
        x1 = x1 ^ x0
    x0 = x0 + ks[1]
    x1 = x1 + (ks[2] + jnp.uint32(1))
    for g in range(1, 5):
        for r in rot[g % 2]:
            x0 = x0 + x1
            x1 = _rotl(x1, r)
            x1 = x1 ^ x0
        x0 = x0 + ks[(g + 1) % 3]
        x1 = x1 + (ks[(g + 2) % 3] + jnp.uint32(g + 1))
    return x0 ^ x1


def _body(logits_ref, out_ref):
    i = pl.program_id(0)
    base0 = (i * jnp.int32(_R * _K)).astype(jnp.uint32)
    row_u = jax.lax.broadcasted_iota(jnp.uint32, (_R, _C), 0)
    col_u = jax.lax.broadcasted_iota(jnp.uint32, (_R, _C), 1)
    cnt0 = base0 + row_u * jnp.uint32(_K) + col_u
    col0 = jax.lax.broadcasted_iota(jnp.int32, (_R, _C), 1)

    m = jnp.full((_R, 1), -jnp.inf, dtype=jnp.float32)
    idx = jnp.zeros((_R, 1), dtype=jnp.int32)
    for c in range(_NC):
        cnt = cnt0 + jnp.uint32(c * _C)
        bits = _threefry_bits(cnt)
        fb = (bits >> jnp.uint32(9)) | jnp.uint32(0x3F800000)
        f = jax.lax.bitcast_convert_type(fb, jnp.float32) - jnp.float32(1.0)
        u = f + _TINY
        g = -jnp.log(-jnp.log(u))

        p = logits_ref[:, c * _C:(c + 1) * _C] + g
        cm = jnp.max(p, axis=1, keepdims=True)
        cidx = jnp.min(jnp.where(p == cm, col0 + jnp.int32(c * _C),
                                 jnp.int32(_K)), axis=1, keepdims=True)
        take = cm > m
        m = jnp.maximum(m, cm)
        idx = jnp.where(take, cidx, idx)

    for c in range(_NC):
        col = col0 + jnp.int32(c * _C)
        out_ref[:, c * _C:(c + 1) * _C] = (col == idx).astype(jnp.float32)


def kernel(logits):
    flat = logits.reshape(_M, _K)
    out = pl.pallas_call(
        _body,
        grid=(_M // _R,),
        in_specs=[pl.BlockSpec((_R, _K), lambda i: (i, 0))],
        out_specs=pl.BlockSpec((_R, _K), lambda i: (i, 0)),
        out_shape=jax.ShapeDtypeStruct((_M, _K), jnp.float32),
        compiler_params=pltpu.CompilerParams(
            dimension_semantics=("parallel",),
        ),
    )(flat)
    return out.reshape(logits.shape)


# speedup vs baseline: 1.0204x; 1.0204x over previous
import jax
import jax.numpy as jnp
import numpy as np
from jax.experimental import pallas as pl
from jax.experimental.pallas import tpu as pltpu

_M = 1024
_K = 32768
_R = 8
_C = 1024
_NC = _K // _C
_TINY = np.float32(np.finfo(np.float32).tiny)


def _rotl(x, r):
    return (x << jnp.uint32(r)) | (x >> jnp.uint32(32 - r))


def _threefry_bits(cnt):
    k0 = jnp.uint32(0)
    k1 = jnp.uint32(42)
    k2 = k0 ^ k1 ^ jnp.uint32(0x1BD11BDA)
    ks = (k0, k1, k2)
    rot = ((13, 15, 26, 6), (17, 29, 16, 24))
    x1 = cnt + ks[1]
    x0 = x1
    x1 = _rotl(x1, 13)
    x1 = x1 ^ x0
    for r in (15, 26, 6):
        x0 = x0 + x1
        x1 = _rotl(x1, r)
        x1 = x1 ^ x0
    x0 = x0 + ks[1]
    x1 = x1 + (ks[2] + jnp.uint32(1))
    for g in range(1, 5):
        for r in rot[g % 2]:
            x0 = x0 + x1
            x1 = _rotl(x1, r)
            x1 = x1 ^ x0
        x0 = x0 + ks[(g + 1) % 3]
        x1 = x1 + (ks[(g + 2) % 3] + jnp.uint32(g + 1))
    return x0 ^ x1


def _body(logits_ref, out_ref):
    i = pl.program_id(0)
    base0 = (i * jnp.int32(_R * _K)).astype(jnp.uint32)
    row_u = jax.lax.broadcasted_iota(jnp.uint32, (_R, _C), 0)
    col_u = jax.lax.broadcasted_iota(jnp.uint32, (_R, _C), 1)
    cnt0 = base0 + row_u * jnp.uint32(_K) + col_u
    col0 = jax.lax.broadcasted_iota(jnp.int32, (_R, _C), 1)

    m = jnp.full((_R, 1), -jnp.inf, dtype=jnp.float32)
    icnt = jnp.zeros((_R, 1), dtype=jnp.int32)
    for c in range(_NC):
        cnt = cnt0 + jnp.uint32(c * _C)
        bits = _threefry_bits(cnt)
        fb = (bits >> jnp.uint32(9)) | jnp.uint32(0x3F800000)
        f = jax.lax.bitcast_convert_type(fb, jnp.float32) - jnp.float32(1.0)
        u = f + _TINY
        g = -jnp.log(-jnp.log(u))

        p = logits_ref[:, c * _C:(c + 1) * _C] + g
        cm = jnp.max(p, axis=1, keepdims=True)
        ccnt = jnp.min(jnp.where(p == cm,
                                 jax.lax.bitcast_convert_type(cnt, jnp.int32),
                                 jnp.int32(0x7FFFFFFF)),
                       axis=1, keepdims=True)
        take = cm > m
        m = jnp.maximum(m, cm)
        icnt = jnp.where(take, ccnt, icnt)

    row1 = jax.lax.broadcasted_iota(jnp.int32, (_R, 1), 0)
    idx = icnt - jax.lax.bitcast_convert_type(base0, jnp.int32) - row1 * _K

    for c in range(_NC):
        idx_c = idx - jnp.int32(c * _C)
        out_ref[:, c * _C:(c + 1) * _C] = (col0 == idx_c).astype(jnp.float32)


def kernel(logits):
    flat = logits.reshape(_M, _K)
    out = pl.pallas_call(
        _body,
        grid=(_M // _R,),
        in_specs=[pl.BlockSpec((_R, _K), lambda i: (i, 0))],
        out_specs=pl.BlockSpec((_R, _K), lambda i: (i, 0)),
        out_shape=jax.ShapeDtypeStruct((_M, _K), jnp.float32),
        compiler_params=pltpu.CompilerParams(
            dimension_semantics=("parallel",),
        ),
    )(flat)
    return out.reshape(logits.shape)


# speedup vs baseline: 1.0322x; 1.0116x over previous
import jax
import jax.numpy as jnp
import numpy as np
from jax.experimental import pallas as pl
from jax.experimental.pallas import tpu as pltpu

_M = 1024
_K = 32768
_R = 16
_C = 512
_NC = _K // _C
_TINY = np.float32(np.finfo(np.float32).tiny)


def _rotl(x, r):
    return (x << jnp.uint32(r)) | (x >> jnp.uint32(32 - r))


def _threefry_bits(w):
    k0 = jnp.uint32(0)
    k1 = jnp.uint32(42)
    k2 = k0 ^ k1 ^ jnp.uint32(0x1BD11BDA)
    ks = (k0, k1, k2)
    rot = ((13, 15, 26, 6), (17, 29, 16, 24))
    x1 = w
    x0 = x1
    x1 = _rotl(x1, 13)
    x1 = x1 ^ x0
    for r in (15, 26, 6):
        x0 = x0 + x1
        x1 = _rotl(x1, r)
        x1 = x1 ^ x0
    x0 = x0 + ks[1]
    x1 = x1 + (ks[2] + jnp.uint32(1))
    for g in range(1, 5):
        for r in rot[g % 2]:
            x0 = x0 + x1
            x1 = _rotl(x1, r)
            x1 = x1 ^ x0
        x0 = x0 + ks[(g + 1) % 3]
        x1 = x1 + (ks[(g + 2) % 3] + jnp.uint32(g + 1))
    return x0 ^ x1


def _body(logits_ref, out_ref):
    i = pl.program_id(0)
    base0 = (i * jnp.int32(_R * _K)).astype(jnp.uint32)
    row_u = jax.lax.broadcasted_iota(jnp.uint32, (_R, _C), 0)
    col_u = jax.lax.broadcasted_iota(jnp.uint32, (_R, _C), 1)
    w = base0 + row_u * jnp.uint32(_K) + col_u + jnp.uint32(42)
    col0 = jax.lax.broadcasted_iota(jnp.int32, (_R, _C), 1)

    m = jnp.full((_R, 1), -jnp.inf, dtype=jnp.float32)
    icnt = jnp.zeros((_R, 1), dtype=jnp.int32)
    for c in range(_NC):
        bits = _threefry_bits(w)
        fb = (bits >> jnp.uint32(9)) | jnp.uint32(0x3F800000)
        f = jax.lax.bitcast_convert_type(fb, jnp.float32) - jnp.float32(1.0)
        u = f + _TINY
        g = -jnp.log(-jnp.log(u))

        p = logits_ref[:, c * _C:(c + 1) * _C] + g
        cm = jnp.max(p, axis=1, keepdims=True)
        ccnt = jnp.min(jnp.where(p == cm,
                                 jax.lax.bitcast_convert_type(w, jnp.int32),
                                 jnp.int32(0x7FFFFFFF)),
                       axis=1, keepdims=True)
        take = cm > m
        m = jnp.maximum(m, cm)
        icnt = jnp.where(take, ccnt, icnt)
        w = w + jnp.uint32(_C)

    row1 = jax.lax.broadcasted_iota(jnp.int32, (_R, 1), 0)
    idx = (icnt - jnp.int32(42)
           - jax.lax.bitcast_convert_type(base0, jnp.int32) - row1 * _K)

    for c in range(_NC):
        idx_c = idx - jnp.int32(c * _C)
        out_ref[:, c * _C:(c + 1) * _C] = (col0 == idx_c).astype(jnp.float32)


def kernel(logits):
    flat = logits.reshape(_M, _K)
    out = pl.pallas_call(
        _body,
        grid=(_M // _R,),
        in_specs=[pl.BlockSpec((_R, _K), lambda i: (i, 0))],
        out_specs=pl.BlockSpec((_R, _K), lambda i: (i, 0)),
        out_shape=jax.ShapeDtypeStruct((_M, _K), jnp.float32),
        compiler_params=pltpu.CompilerParams(
            dimension_semantics=("parallel",),
        ),
    )(flat)
    return out.reshape(logits.shape)
